# baseline (speedup 1.0000x reference)
import functools

import jax
import jax.numpy as jnp
from jax import lax
from jax.experimental import pallas as pl
from jax.experimental.pallas import tpu as pltpu
from jax.experimental.pallas import tpu_sc as plsc

N_NODES = 10000
N_EDGES = 320000
D = 128

NC = 2
NS = 16
NW = NC * NS

K = 128
E_PAD = 327680
CHUNKS_PER_TILE = E_PAD // (NW * K)
N_PAD = 10112
ROWS_PER_TILE = N_PAD // NS

WIN = 8
NSLOT = 2

CH0 = 2 * CHUNKS_PER_TILE
NWIN0 = CH0 // WIN


def _seg_sum_body(x_hbm, src_hbm, dst_hbm, zeros_hbm, out_hbm,
                  acc, sw0, sw1, dw0, dw1, b0, b1,
                  iws0, iws1, iwd0, iwd1, g0, g1, s0, s1):
    swin = (sw0, sw1)
    dwin = (dw0, dw1)
    bufs = (b0, b1)
    iwsem_s = (iws0, iws1)
    iwsem_d = (iwd0, iwd1)
    gsem = (g0, g1)
    ssem = (s0, s1)
    c = lax.axis_index("c")
    s = lax.axis_index("s")
    chunk0 = s * CH0
    nwin = NWIN0

    @pl.when(c == 0)
    def _core0_body():
        _core0(x_hbm, src_hbm, dst_hbm, zeros_hbm, out_hbm, acc, swin, dwin,
               bufs, iwsem_s, iwsem_d, gsem, ssem, s, chunk0, nwin)


def _core0(x_hbm, src_hbm, dst_hbm, zeros_hbm, out_hbm, acc, swin, dwin,
           bufs, iwsem_s, iwsem_d, gsem, ssem, s, chunk0, nwin):
    rows0 = s * ROWS_PER_TILE
    pltpu.sync_copy(zeros_hbm.at[pl.ds(rows0, ROWS_PER_TILE)],
                    acc.at[pl.ds(rows0, ROWS_PER_TILE)])

    def fire_win(w, slot):
        base = chunk0 + w * WIN
        pltpu.async_copy(src_hbm.at[pl.ds(base, WIN)], swin[slot],
                         iwsem_s[slot])
        pltpu.async_copy(dst_hbm.at[pl.ds(base, WIN)], dwin[slot],
                         iwsem_d[slot])

    def wait_win(slot):
        pltpu.make_async_copy(src_hbm.at[pl.ds(0, WIN)], swin[slot],
                              iwsem_s[slot]).wait()
        pltpu.make_async_copy(dst_hbm.at[pl.ds(0, WIN)], dwin[slot],
                              iwsem_d[slot]).wait()

    fire_win(0, 0)
    fire_win(1, 1)
    wait_win(0)
    pltpu.async_copy(x_hbm.at[swin[0].at[0]], bufs[0], gsem[0])
    pltpu.async_copy(x_hbm.at[swin[0].at[1]], bufs[1], gsem[1])
    plsc.subcore_barrier()

    def window(u, carry):
        for ws in range(2):
            w = u * 2 + ws
            _one_window(w, ws)
        return carry

    def _one_window(w, ws):
        for r in range(WIN):
            b = r % NSLOT
            pltpu.make_async_copy(x_hbm.at[swin[ws].at[r]], bufs[b],
                                  gsem[b]).wait()
            pltpu.async_copy(bufs[b], acc.at[dwin[ws].at[r]], ssem[b],
                             add=True)
            if r < WIN - NSLOT:
                pltpu.make_async_copy(bufs[b], acc.at[dwin[ws].at[r]],
                                      ssem[b]).wait()
                pltpu.async_copy(x_hbm.at[swin[ws].at[r + NSLOT]], bufs[b],
                                 gsem[b])
            else:
                if r == WIN - NSLOT:
                    @pl.when(w < nwin - 1)
                    def _():
                        wait_win(1 - ws)

                @pl.when(w < nwin - 1)
                def _():
                    pltpu.make_async_copy(bufs[b], acc.at[dwin[ws].at[r]],
                                          ssem[b]).wait()
                    pltpu.async_copy(
                        x_hbm.at[swin[1 - ws].at[r - (WIN - NSLOT)]],
                        bufs[b], gsem[b])

        @pl.when(w < nwin - 2)
        def _():
            fire_win(w + 2, ws)

    lax.fori_loop(0, nwin // 2, window, 0, unroll=False)

    for b in range(NSLOT):
        r = WIN - NSLOT + b
        pltpu.make_async_copy(bufs[b], acc.at[dwin[1].at[r]], ssem[b]).wait()
    plsc.subcore_barrier()

    pltpu.sync_copy(acc.at[pl.ds(rows0, ROWS_PER_TILE)],
                    out_hbm.at[pl.ds(rows0, ROWS_PER_TILE)])


_seg_sum = pl.kernel(
    _seg_sum_body,
    out_type=jax.ShapeDtypeStruct((N_PAD, D), jnp.float32),
    mesh=plsc.VectorSubcoreMesh(core_axis_name="c", subcore_axis_name="s",
                                num_cores=NC, num_subcores=NS),
    scratch_types=[
        pltpu.VMEM_SHARED((N_PAD, D), jnp.float32),
    ] + [pltpu.VMEM((WIN, K), jnp.int32) for _ in range(4)]
      + [pltpu.VMEM((K, D), jnp.float32) for _ in range(NSLOT)]
      + [pltpu.SemaphoreType.DMA for _ in range(4 + 2 * NSLOT)],
)


def _linear_body(relu, acc_ref, w_ref, b_ref, o_ref):
    x = acc_ref[...]
    y = jnp.dot(x, w_ref[...], preferred_element_type=jnp.float32,
                precision=lax.Precision.HIGHEST)
    y = y + b_ref[...]
    if relu:
        y = jnp.maximum(y, 0.0)
    o_ref[...] = y


def _linear(acc, w, b, relu):
    blk = 632
    return pl.pallas_call(
        functools.partial(_linear_body, relu),
        grid=(N_PAD // blk,),
        in_specs=[
            pl.BlockSpec((blk, D), lambda i: (i, 0)),
            pl.BlockSpec((D, D), lambda i: (0, 0)),
            pl.BlockSpec((1, D), lambda i: (0, 0)),
        ],
        out_specs=pl.BlockSpec((blk, D), lambda i: (i, 0)),
        out_shape=jax.ShapeDtypeStruct((N_PAD, D), jnp.float32),
    )(acc, w, b)


def kernel(feature, edge_index, W1, b1, W2, b2):
    ei = edge_index.astype(jnp.int32)
    src = jnp.concatenate(
        [ei[0], jnp.zeros((E_PAD - N_EDGES,), jnp.int32)]).reshape(-1, K)
    dst = jnp.concatenate(
        [ei[1], jnp.full((E_PAD - N_EDGES,), N_NODES, jnp.int32)]).reshape(-1, K)
    zeros = jnp.zeros((N_PAD, D), jnp.float32)
    x = jnp.concatenate(
        [feature, jnp.zeros((N_PAD - N_NODES, D), jnp.float32)])

    acc1 = _seg_sum(x, src, dst, zeros)
    h = _linear(acc1, W1, b1.reshape(1, D), relu=True)
    acc2 = _seg_sum(h, src, dst, zeros)
    out = _linear(acc2, W2, b2.reshape(1, D), relu=False)
    return out[:N_NODES]


# speedup vs baseline: 3.2485x; 3.2485x over previous
import functools

import jax
import jax.numpy as jnp
from jax import lax
from jax.experimental import pallas as pl
from jax.experimental.pallas import tpu as pltpu
from jax.experimental.pallas import tpu_sc as plsc

N_NODES = 10000
N_EDGES = 320000
D = 128

NC = 2
NS = 16
NW = NC * NS

K = 128
E_PAD = 327680
CHUNKS_PER_TILE = E_PAD // (NW * K)
N_PAD = 10112
ROWS_PER_TILE = N_PAD // NS

WIN = 8
NSLOT = 2

NWINDOWS = CHUNKS_PER_TILE // WIN


def _seg_sum_body(x_hbm, src_hbm, dst_hbm, zeros_hbm, out_hbm,
                  acc, sw0, sw1, dw0, dw1, b0, b1,
                  iws0, iws1, iwd0, iwd1, g0, g1, s0, s1):
    swin = (sw0, sw1)
    dwin = (dw0, dw1)
    bufs = (b0, b1)
    iwsem_s = (iws0, iws1)
    iwsem_d = (iwd0, iwd1)
    gsem = (g0, g1)
    ssem = (s0, s1)
    c = lax.axis_index("c")
    s = lax.axis_index("s")
    chunk0 = (c * NS + s) * CHUNKS_PER_TILE
    nwin = NWINDOWS

    rows0 = s * ROWS_PER_TILE
    pltpu.sync_copy(zeros_hbm.at[pl.ds(rows0, ROWS_PER_TILE)],
                    acc.at[pl.ds(rows0, ROWS_PER_TILE)])

    def fire_win(w, slot):
        base = chunk0 + w * WIN
        pltpu.async_copy(src_hbm.at[pl.ds(base, WIN)], swin[slot],
                         iwsem_s[slot])
        pltpu.async_copy(dst_hbm.at[pl.ds(base, WIN)], dwin[slot],
                         iwsem_d[slot])

    def wait_win(slot):
        pltpu.make_async_copy(src_hbm.at[pl.ds(0, WIN)], swin[slot],
                              iwsem_s[slot]).wait()
        pltpu.make_async_copy(dst_hbm.at[pl.ds(0, WIN)], dwin[slot],
                              iwsem_d[slot]).wait()

    fire_win(0, 0)
    fire_win(1, 1)
    wait_win(0)
    pltpu.async_copy(x_hbm.at[swin[0].at[0]], bufs[0], gsem[0])
    pltpu.async_copy(x_hbm.at[swin[0].at[1]], bufs[1], gsem[1])
    plsc.subcore_barrier()

    def window(u, carry):
        for ws in range(2):
            w = u * 2 + ws
            _one_window(w, ws)
        return carry

    def _one_window(w, ws):
        for r in range(WIN):
            b = r % NSLOT
            pltpu.make_async_copy(x_hbm.at[swin[ws].at[r]], bufs[b],
                                  gsem[b]).wait()
            pltpu.async_copy(bufs[b], acc.at[dwin[ws].at[r]], ssem[b],
                             add=True)
            if r < WIN - NSLOT:
                pltpu.make_async_copy(bufs[b], acc.at[dwin[ws].at[r]],
                                      ssem[b]).wait()
                pltpu.async_copy(x_hbm.at[swin[ws].at[r + NSLOT]], bufs[b],
                                 gsem[b])
            else:
                if r == WIN - NSLOT:
                    @pl.when(w < nwin - 1)
                    def _():
                        wait_win(1 - ws)

                @pl.when(w < nwin - 1)
                def _():
                    pltpu.make_async_copy(bufs[b], acc.at[dwin[ws].at[r]],
                                          ssem[b]).wait()
                    pltpu.async_copy(
                        x_hbm.at[swin[1 - ws].at[r - (WIN - NSLOT)]],
                        bufs[b], gsem[b])

        @pl.when(w < nwin - 2)
        def _():
            fire_win(w + 2, ws)

    lax.fori_loop(0, nwin // 2, window, 0, unroll=False)

    for b in range(NSLOT):
        r = WIN - NSLOT + b
        pltpu.make_async_copy(bufs[b], acc.at[dwin[1].at[r]], ssem[b]).wait()
    plsc.subcore_barrier()

    pltpu.sync_copy(acc.at[pl.ds(rows0, ROWS_PER_TILE)],
                    out_hbm.at[pl.ds(c * N_PAD + rows0, ROWS_PER_TILE)])


_seg_sum = pl.kernel(
    _seg_sum_body,
    out_type=jax.ShapeDtypeStruct((NC * N_PAD, D), jnp.float32),
    mesh=plsc.VectorSubcoreMesh(core_axis_name="c", subcore_axis_name="s",
                                num_cores=NC, num_subcores=NS),
    scratch_types=[
        pltpu.VMEM_SHARED((N_PAD, D), jnp.float32),
    ] + [pltpu.VMEM((WIN, K), jnp.int32) for _ in range(4)]
      + [pltpu.VMEM((K, D), jnp.float32) for _ in range(NSLOT)]
      + [pltpu.SemaphoreType.DMA for _ in range(4 + 2 * NSLOT)],
)


def _linear_body(relu, acc_ref, w_ref, b_ref, o_ref):
    x = acc_ref[0] + acc_ref[1]
    y = jnp.dot(x, w_ref[...], preferred_element_type=jnp.float32,
                precision=lax.Precision.HIGHEST)
    y = y + b_ref[...]
    if relu:
        y = jnp.maximum(y, 0.0)
    o_ref[...] = y


def _linear(acc, w, b, relu):
    blk = 632
    return pl.pallas_call(
        functools.partial(_linear_body, relu),
        grid=(N_PAD // blk,),
        in_specs=[
            pl.BlockSpec((NC, blk, D), lambda i: (0, i, 0)),
            pl.BlockSpec((D, D), lambda i: (0, 0)),
            pl.BlockSpec((1, D), lambda i: (0, 0)),
        ],
        out_specs=pl.BlockSpec((blk, D), lambda i: (i, 0)),
        out_shape=jax.ShapeDtypeStruct((N_PAD, D), jnp.float32),
    )(acc, w, b)


def kernel(feature, edge_index, W1, b1, W2, b2):
    ei = edge_index.astype(jnp.int32)
    n_pad_e = E_PAD - N_EDGES
    pad_iota = jnp.arange(n_pad_e, dtype=jnp.int32)
    src = jnp.concatenate(
        [ei[0], pad_iota % N_NODES]).reshape(-1, K)
    dst = jnp.concatenate(
        [ei[1], N_NODES + pad_iota % (N_PAD - N_NODES)]).reshape(-1, K)
    zeros = jnp.zeros((N_PAD, D), jnp.float32)
    x = jnp.concatenate(
        [feature, jnp.zeros((N_PAD - N_NODES, D), jnp.float32)])

    acc1 = _seg_sum(x, src, dst, zeros).reshape(NC, N_PAD, D)
    h = _linear(acc1, W1, b1.reshape(1, D), relu=True)
    acc2 = _seg_sum(h, src, dst, zeros).reshape(NC, N_PAD, D)
    out = _linear(acc2, W2, b2.reshape(1, D), relu=False)
    return out[:N_NODES]


# speedup vs baseline: 3.2675x; 1.0058x over previous
import functools

import jax
import jax.numpy as jnp
from jax import lax
from jax.experimental import pallas as pl
from jax.experimental.pallas import tpu as pltpu
from jax.experimental.pallas import tpu_sc as plsc

N_NODES = 10000
N_EDGES = 320000
D = 128

NC = 2
NS = 16
NW = NC * NS

K = 128
E_PAD = 327680
CHUNKS_PER_TILE = E_PAD // (NW * K)
N_PAD = 10112
ROWS_PER_TILE = N_PAD // NS

WIN = 8
NSLOT = 2

NWINDOWS = CHUNKS_PER_TILE // WIN


def _seg_sum_body(x_hbm, src_hbm, dst_hbm, zeros_hbm, out_hbm,
                  acc, sw0, sw1, dw0, dw1, b0, b1,
                  iws0, iws1, iwd0, iwd1, g0, g1, s0, s1):
    swin = (sw0, sw1)
    dwin = (dw0, dw1)
    bufs = (b0, b1)
    iwsem_s = (iws0, iws1)
    iwsem_d = (iwd0, iwd1)
    gsem = (g0, g1)
    ssem = (s0, s1)
    c = lax.axis_index("c")
    s = lax.axis_index("s")
    chunk0 = (c * NS + s) * CHUNKS_PER_TILE
    nwin = NWINDOWS

    rows0 = s * ROWS_PER_TILE
    pltpu.sync_copy(zeros_hbm.at[pl.ds(rows0, ROWS_PER_TILE)],
                    acc.at[pl.ds(rows0, ROWS_PER_TILE)])

    def fire_win(w, slot):
        base = chunk0 + w * WIN
        pltpu.async_copy(src_hbm.at[pl.ds(base, WIN)], swin[slot],
                         iwsem_s[slot])
        pltpu.async_copy(dst_hbm.at[pl.ds(base, WIN)], dwin[slot],
                         iwsem_d[slot])

    def wait_win(slot):
        pltpu.make_async_copy(src_hbm.at[pl.ds(0, WIN)], swin[slot],
                              iwsem_s[slot]).wait()
        pltpu.make_async_copy(dst_hbm.at[pl.ds(0, WIN)], dwin[slot],
                              iwsem_d[slot]).wait()

    fire_win(0, 0)
    fire_win(1, 1)
    wait_win(0)
    pltpu.async_copy(x_hbm.at[swin[0].at[0]], bufs[0], gsem[0])
    pltpu.async_copy(x_hbm.at[swin[0].at[1]], bufs[1], gsem[1])
    plsc.subcore_barrier()

    def window(u, carry):
        for ws in range(2):
            w = u * 2 + ws
            _one_window(w, ws)
        return carry

    def _one_window(w, ws):
        for r in range(WIN):
            b = r % NSLOT
            pltpu.make_async_copy(x_hbm.at[swin[ws].at[r]], bufs[b],
                                  gsem[b]).wait()
            pltpu.async_copy(bufs[b], acc.at[dwin[ws].at[r]], ssem[b],
                             add=True)
            if r < WIN - NSLOT:
                pltpu.make_async_copy(bufs[b], acc.at[dwin[ws].at[r]],
                                      ssem[b]).wait()
                pltpu.async_copy(x_hbm.at[swin[ws].at[r + NSLOT]], bufs[b],
                                 gsem[b])
            else:
                if r == WIN - NSLOT:
                    @pl.when(w < nwin - 1)
                    def _():
                        wait_win(1 - ws)

                @pl.when(w < nwin - 1)
                def _():
                    pltpu.make_async_copy(bufs[b], acc.at[dwin[ws].at[r]],
                                          ssem[b]).wait()
                    pltpu.async_copy(
                        x_hbm.at[swin[1 - ws].at[r - (WIN - NSLOT)]],
                        bufs[b], gsem[b])

        @pl.when(w < nwin - 2)
        def _():
            fire_win(w + 2, ws)

    lax.fori_loop(0, nwin // 2, window, 0, unroll=False)

    for b in range(NSLOT):
        r = WIN - NSLOT + b
        pltpu.make_async_copy(bufs[b], acc.at[dwin[1].at[r]], ssem[b]).wait()
    plsc.subcore_barrier()

    pltpu.sync_copy(acc.at[pl.ds(rows0, ROWS_PER_TILE)],
                    out_hbm.at[pl.ds(c * N_PAD + rows0, ROWS_PER_TILE)])


def _make_seg_sum():
    return pl.kernel(
        _seg_sum_body,
        out_type=jax.ShapeDtypeStruct((NC * N_PAD, D), jnp.float32),
        mesh=plsc.VectorSubcoreMesh(core_axis_name="c", subcore_axis_name="s",
                                    num_cores=NC, num_subcores=NS),
        scratch_types=[
            pltpu.VMEM_SHARED((N_PAD, D), jnp.float32),
        ] + [pltpu.VMEM((WIN, K), jnp.int32) for _ in range(4)]
          + [pltpu.VMEM((K, D), jnp.float32) for _ in range(NSLOT)]
          + [pltpu.SemaphoreType.DMA for _ in range(4 + 2 * NSLOT)],
    )


_seg_sum = _make_seg_sum()


def _linear_body(relu, acc_ref, w_ref, b_ref, o_ref):
    x = acc_ref[0] + acc_ref[1]
    y = jnp.dot(x, w_ref[...], preferred_element_type=jnp.float32,
                precision=lax.Precision.HIGHEST)
    y = y + b_ref[...]
    if relu:
        y = jnp.maximum(y, 0.0)
    o_ref[...] = y


def _linear(acc, w, b, relu):
    blk = 632
    return pl.pallas_call(
        functools.partial(_linear_body, relu),
        grid=(N_PAD // blk,),
        in_specs=[
            pl.BlockSpec((NC, blk, D), lambda i: (0, i, 0)),
            pl.BlockSpec((D, D), lambda i: (0, 0)),
            pl.BlockSpec((1, D), lambda i: (0, 0)),
        ],
        out_specs=pl.BlockSpec((blk, D), lambda i: (i, 0)),
        out_shape=jax.ShapeDtypeStruct((N_PAD, D), jnp.float32),
    )(acc, w, b)


def kernel(feature, edge_index, W1, b1, W2, b2):
    ei = edge_index.astype(jnp.int32)
    n_pad_e = E_PAD - N_EDGES
    pad_iota = jnp.arange(n_pad_e, dtype=jnp.int32)
    src = jnp.concatenate(
        [ei[0], pad_iota % N_NODES]).reshape(-1, K)
    dst = jnp.concatenate(
        [ei[1], N_NODES + pad_iota % (N_PAD - N_NODES)]).reshape(-1, K)
    zeros = jnp.zeros((N_PAD, D), jnp.float32)

    acc1 = _seg_sum(feature, src, dst, zeros).reshape(NC, N_PAD, D)
    h = _linear(acc1, W1, b1.reshape(1, D), relu=True)
    acc2 = _seg_sum(h, src, dst, zeros).reshape(NC, N_PAD, D)
    out = _linear(acc2, W2, b2.reshape(1, D), relu=False)
    return out[:N_NODES]


# speedup vs baseline: 3.2730x; 1.0017x over previous
import functools

import jax
import jax.numpy as jnp
import numpy as np
from jax import lax
from jax.experimental import pallas as pl
from jax.experimental.pallas import tpu as pltpu
from jax.experimental.pallas import tpu_sc as plsc

N_NODES = 10000
N_EDGES = 320000
D = 128

NC = 2
NS = 16
NW = NC * NS

K = 128
E_PAD = 327680
CHUNKS_PER_TILE = E_PAD // (NW * K)
N_PAD = 10112
ROWS_PER_TILE = N_PAD // NS

WIN = 8
NSLOT = 2

NWINDOWS = CHUNKS_PER_TILE // WIN


def _seg_sum_body(x_hbm, src_hbm, dst_hbm, zeros_hbm, out_hbm,
                  acc, sw0, sw1, dw0, dw1, b0, b1,
                  iws0, iws1, iwd0, iwd1, g0, g1, s0, s1):
    swin = (sw0, sw1)
    dwin = (dw0, dw1)
    bufs = (b0, b1)
    iwsem_s = (iws0, iws1)
    iwsem_d = (iwd0, iwd1)
    gsem = (g0, g1)
    ssem = (s0, s1)
    c = lax.axis_index("c")
    s = lax.axis_index("s")
    chunk0 = (c * NS + s) * CHUNKS_PER_TILE
    nwin = NWINDOWS

    rows0 = s * ROWS_PER_TILE
    pltpu.sync_copy(zeros_hbm.at[pl.ds(rows0, ROWS_PER_TILE)],
                    acc.at[pl.ds(rows0, ROWS_PER_TILE)])

    def fire_win(w, slot):
        base = chunk0 + w * WIN
        pltpu.async_copy(src_hbm.at[pl.ds(base, WIN)], swin[slot],
                         iwsem_s[slot])
        pltpu.async_copy(dst_hbm.at[pl.ds(base, WIN)], dwin[slot],
                         iwsem_d[slot])

    def wait_win(slot):
        pltpu.make_async_copy(src_hbm.at[pl.ds(0, WIN)], swin[slot],
                              iwsem_s[slot]).wait()
        pltpu.make_async_copy(dst_hbm.at[pl.ds(0, WIN)], dwin[slot],
                              iwsem_d[slot]).wait()

    fire_win(0, 0)
    fire_win(1, 1)
    wait_win(0)
    pltpu.async_copy(x_hbm.at[swin[0].at[0]], bufs[0], gsem[0])
    pltpu.async_copy(x_hbm.at[swin[0].at[1]], bufs[1], gsem[1])
    plsc.subcore_barrier()

    def window(u, carry):
        for ws in range(2):
            w = u * 2 + ws
            _one_window(w, ws)
        return carry

    def _one_window(w, ws):
        for r in range(WIN):
            b = r % NSLOT
            pltpu.make_async_copy(x_hbm.at[swin[ws].at[r]], bufs[b],
                                  gsem[b]).wait()
            pltpu.async_copy(bufs[b], acc.at[dwin[ws].at[r]], ssem[b],
                             add=True)
            if r < WIN - NSLOT:
                pltpu.make_async_copy(bufs[b], acc.at[dwin[ws].at[r]],
                                      ssem[b]).wait()
                pltpu.async_copy(x_hbm.at[swin[ws].at[r + NSLOT]], bufs[b],
                                 gsem[b])
            else:
                if r == WIN - NSLOT:
                    @pl.when(w < nwin - 1)
                    def _():
                        wait_win(1 - ws)

                @pl.when(w < nwin - 1)
                def _():
                    pltpu.make_async_copy(bufs[b], acc.at[dwin[ws].at[r]],
                                          ssem[b]).wait()
                    pltpu.async_copy(
                        x_hbm.at[swin[1 - ws].at[r - (WIN - NSLOT)]],
                        bufs[b], gsem[b])

        @pl.when(w < nwin - 2)
        def _():
            fire_win(w + 2, ws)

    lax.fori_loop(0, nwin // 2, window, 0, unroll=False)

    for b in range(NSLOT):
        r = WIN - NSLOT + b
        pltpu.make_async_copy(bufs[b], acc.at[dwin[1].at[r]], ssem[b]).wait()
    plsc.subcore_barrier()

    pltpu.sync_copy(acc.at[pl.ds(rows0, ROWS_PER_TILE)],
                    out_hbm.at[pl.ds(c * N_PAD + rows0, ROWS_PER_TILE)])


def _make_seg_sum():
    return pl.kernel(
        _seg_sum_body,
        out_type=jax.ShapeDtypeStruct((NC * N_PAD, D), jnp.float32),
        mesh=plsc.VectorSubcoreMesh(core_axis_name="c", subcore_axis_name="s",
                                    num_cores=NC, num_subcores=NS),
        scratch_types=[
            pltpu.VMEM_SHARED((N_PAD, D), jnp.float32),
        ] + [pltpu.VMEM((WIN, K), jnp.int32) for _ in range(4)]
          + [pltpu.VMEM((K, D), jnp.float32) for _ in range(NSLOT)]
          + [pltpu.SemaphoreType.DMA for _ in range(4 + 2 * NSLOT)],
    )


_seg_sum = _make_seg_sum()


def _linear_body(relu, acc_ref, w_ref, b_ref, o_ref):
    x = acc_ref[0] + acc_ref[1]
    y = jnp.dot(x, w_ref[...], preferred_element_type=jnp.float32,
                precision=lax.Precision.HIGHEST)
    y = y + b_ref[...]
    if relu:
        y = jnp.maximum(y, 0.0)
    o_ref[...] = y


def _linear(acc, w, b, relu):
    blk = 400
    return pl.pallas_call(
        functools.partial(_linear_body, relu),
        grid=(N_NODES // blk,),
        in_specs=[
            pl.BlockSpec((NC, blk, D), lambda i: (0, i, 0)),
            pl.BlockSpec((D, D), lambda i: (0, 0)),
            pl.BlockSpec((1, D), lambda i: (0, 0)),
        ],
        out_specs=pl.BlockSpec((blk, D), lambda i: (i, 0)),
        out_shape=jax.ShapeDtypeStruct((N_NODES, D), jnp.float32),
    )(acc, w, b)


_PAD_SRC = jnp.asarray(np.arange(E_PAD - N_EDGES) % N_NODES, jnp.int32)
_PAD_DST = jnp.asarray(
    N_NODES + np.arange(E_PAD - N_EDGES) % (N_PAD - N_NODES), jnp.int32)


def kernel(feature, edge_index, W1, b1, W2, b2):
    ei = edge_index.astype(jnp.int32)
    src = jnp.concatenate([ei[0], _PAD_SRC]).reshape(-1, K)
    dst = jnp.concatenate([ei[1], _PAD_DST]).reshape(-1, K)
    zeros = jnp.zeros((N_PAD, D), jnp.float32)

    acc1 = _seg_sum(feature, src, dst, zeros).reshape(NC, N_PAD, D)
    h = _linear(acc1, W1, b1.reshape(1, D), relu=True)
    acc2 = _seg_sum(h, src, dst, zeros).reshape(NC, N_PAD, D)
    return _linear(acc2, W2, b2.reshape(1, D), relu=False)
